# Initial kernel scaffold; baseline (speedup 1.0000x reference)
import jax
import jax.numpy as jnp
from jax.experimental import pallas as pl


def kernel(x, edge_index, edge_weight, W1, b1, W2, b2, W3, b3):
    raise NotImplementedError("write your pallas kernel here")


# speedup vs baseline: 6.4012x; 6.4012x over previous
import functools

import jax
import jax.numpy as jnp
from jax import lax
from jax.experimental import pallas as pl
from jax.experimental.pallas import tpu as pltpu
from jax.experimental.pallas import tpu_sc as plsc

_LANES = 16
_SUBCORES = 16
_CORES = 2
_CW = 128


@functools.partial(jax.jit, static_argnums=(0, 1, 2))
def _sc_agg_call(n_chunks, npad, nb, chunks, src2d, dst2d, ew2d):
    rpt = npad // _SUBCORES

    mesh = plsc.VectorSubcoreMesh(
        core_axis_name="c", subcore_axis_name="s",
        num_cores=_CORES, num_subcores=_SUBCORES)

    out_type = [jax.ShapeDtypeStruct((npad, _CW), jnp.float32)
                for _ in range(n_chunks)]
    scratch_types = [
        pltpu.VMEM_SHARED((npad, _CW), jnp.float32),
        pltpu.VMEM((nb, _CW), jnp.int32),
        pltpu.VMEM((nb, _CW), jnp.int32),
        pltpu.VMEM((nb * _CW,), jnp.float32),
        pltpu.VMEM((_CW, _CW), jnp.float32),
    ]

    @functools.partial(pl.kernel, out_type=out_type, mesh=mesh,
                       scratch_types=scratch_types)
    def k(*refs):
        chunk_refs = refs[0:n_chunks]
        src_hbm = refs[n_chunks]
        dst_hbm = refs[n_chunks + 1]
        ew_hbm = refs[n_chunks + 2]
        out_refs = refs[n_chunks + 3: 2 * n_chunks + 3]
        slab, src_v, dst_v, ew_v, rows = refs[2 * n_chunks + 3:]

        cid = lax.axis_index("c")
        sid = lax.axis_index("s")

        pltpu.sync_copy(src_hbm.at[sid], src_v)
        pltpu.sync_copy(dst_hbm.at[sid], dst_v)
        pltpu.sync_copy(ew_hbm.at[sid], ew_v)

        zeros16 = jnp.zeros((_LANES,), jnp.float32)

        for kk in range(n_chunks):
            @pl.when(cid == (kk % _CORES))
            def _():
                def zrow(i, carry):
                    for c in range(_CW // _LANES):
                        rows[i, pl.ds(c * _LANES, _LANES)] = zeros16
                    return carry
                lax.fori_loop(0, _CW, zrow, 0)
                base = sid * rpt
                for t in range(rpt // _CW):
                    pltpu.sync_copy(
                        rows, slab.at[pl.ds(base + t * _CW, _CW)])
                plsc.subcore_barrier()

                def batch(j, carry):
                    pltpu.sync_copy(chunk_refs[kk].at[src_v.at[j]], rows)

                    def group(g, c2):
                        w16 = ew_v[pl.ds(j * _CW + g * _LANES, _LANES)]
                        for i2 in range(_LANES):
                            w = w16[i2]
                            r = g * _LANES + i2
                            for c in range(_CW // _LANES):
                                sl = pl.ds(c * _LANES, _LANES)
                                rows[r, sl] = rows[r, sl] * w
                        return c2
                    lax.fori_loop(0, _CW // _LANES, group, 0)

                    pltpu.sync_copy(rows, slab.at[dst_v.at[j]], add=True)
                    return carry
                lax.fori_loop(0, nb, batch, 0)
                plsc.subcore_barrier()

                pltpu.sync_copy(slab.at[pl.ds(base, rpt)],
                                out_refs[kk].at[pl.ds(base, rpt)])

        return None

    return k(*chunks, src2d, dst2d, ew2d)


def _row_spec(rb, w):
    return pl.BlockSpec((rb, w), lambda i: (i, 0))


def _full_spec(shape):
    return pl.BlockSpec(shape, lambda i: tuple(0 for _ in shape))


def _tc0(degcol, x, rb):
    n, din = x.shape
    nck = din // _CW

    def body(deg_ref, x_ref, dinv_ref, *xp_refs):
        deg = deg_ref[...] + 1.0
        dinv = jnp.where(deg > 0, lax.rsqrt(jnp.maximum(deg, 1e-12)), 0.0)
        dinv_ref[...] = dinv
        xp = x_ref[...] * dinv
        for c in range(nck):
            xp_refs[c][...] = xp[:, c * _CW:(c + 1) * _CW]

    return pl.pallas_call(
        body,
        grid=(n // rb,),
        in_specs=[_row_spec(rb, 1), _row_spec(rb, din)],
        out_specs=[_row_spec(rb, 1)] + [_row_spec(rb, _CW)] * nck,
        out_shape=[jax.ShapeDtypeStruct((n, 1), jnp.float32)]
        + [jax.ShapeDtypeStruct((n, _CW), jnp.float32)] * nck,
    )(degcol, x)


def _tc1(dinv, xps, aggs, W1, b1, W2, rb):
    n = dinv.shape[0]
    nin = len(xps)
    dh = W2.shape[1]
    nout = dh // _CW

    def body(*refs):
        dinv_ref = refs[0]
        xp_refs = refs[1:1 + nin]
        ag_refs = refs[1 + nin:1 + 2 * nin]
        W1_ref, b1_ref, W2_ref = refs[1 + 2 * nin:4 + 2 * nin]
        out_refs = refs[4 + 2 * nin:]
        dinv = dinv_ref[...]
        m1 = jnp.concatenate(
            [ag_refs[c][...] + xp_refs[c][...] for c in range(nin)],
            axis=1) * dinv
        out1 = jnp.maximum(
            jnp.dot(m1, W1_ref[...], preferred_element_type=jnp.float32)
            + b1_ref[...], 0.0)
        h2p = jnp.dot(out1, W2_ref[...],
                      preferred_element_type=jnp.float32) * dinv
        for c in range(nout):
            out_refs[c][...] = h2p[:, c * _CW:(c + 1) * _CW]

    return pl.pallas_call(
        body,
        grid=(n // rb,),
        in_specs=[_row_spec(rb, 1)] + [_row_spec(rb, _CW)] * (2 * nin)
        + [_full_spec(W1.shape), _full_spec(b1.shape), _full_spec(W2.shape)],
        out_specs=[_row_spec(rb, _CW)] * nout,
        out_shape=[jax.ShapeDtypeStruct((n, _CW), jnp.float32)] * nout,
    )(dinv, *xps, *aggs, W1, b1, W2)


def _tc2(dinv, h2ps, aggs, b2, W3, rb):
    n = dinv.shape[0]
    nin = len(h2ps)
    dout = W3.shape[1]
    nout = dout // _CW

    def body(*refs):
        dinv_ref = refs[0]
        hp_refs = refs[1:1 + nin]
        ag_refs = refs[1 + nin:1 + 2 * nin]
        b2_ref, W3_ref = refs[1 + 2 * nin:3 + 2 * nin]
        out_refs = refs[3 + 2 * nin:]
        dinv = dinv_ref[...]
        m2 = jnp.concatenate(
            [ag_refs[c][...] + hp_refs[c][...] for c in range(nin)],
            axis=1) * dinv
        out2 = jnp.maximum(m2 + b2_ref[...], 0.0)
        h3p = jnp.dot(out2, W3_ref[...],
                      preferred_element_type=jnp.float32) * dinv
        for c in range(nout):
            out_refs[c][...] = h3p[:, c * _CW:(c + 1) * _CW]

    return pl.pallas_call(
        body,
        grid=(n // rb,),
        in_specs=[_row_spec(rb, 1)] + [_row_spec(rb, _CW)] * (2 * nin)
        + [_full_spec(b2.shape), _full_spec(W3.shape)],
        out_specs=[_row_spec(rb, _CW)] * nout,
        out_shape=[jax.ShapeDtypeStruct((n, _CW), jnp.float32)] * nout,
    )(dinv, *h2ps, *aggs, b2, W3)


def _tc3(dinv, h3ps, aggs, b3, rb):
    n = dinv.shape[0]
    nin = len(h3ps)
    dout = nin * _CW

    def body(*refs):
        dinv_ref = refs[0]
        hp_refs = refs[1:1 + nin]
        ag_refs = refs[1 + nin:1 + 2 * nin]
        b3_ref = refs[1 + 2 * nin]
        out_ref = refs[2 + 2 * nin]
        dinv = dinv_ref[...]
        m3 = jnp.concatenate(
            [ag_refs[c][...] + hp_refs[c][...] for c in range(nin)],
            axis=1) * dinv
        out_ref[...] = m3 + b3_ref[...]

    return pl.pallas_call(
        body,
        grid=(n // rb,),
        in_specs=[_row_spec(rb, 1)] + [_row_spec(rb, _CW)] * (2 * nin)
        + [_full_spec(b3.shape)],
        out_specs=_row_spec(rb, dout),
        out_shape=jax.ShapeDtypeStruct((n, dout), jnp.float32),
    )(dinv, *h3ps, *aggs, b3)


def kernel(x, edge_index, edge_weight, W1, b1, W2, b2, W3, b3):
    n, din = x.shape
    e = edge_index.shape[1]
    dh = W1.shape[1]
    dout = W3.shape[1]

    rb = 1000 if n % 1000 == 0 else 8 * (n // 8)
    rpt = -(-n // (_SUBCORES * _CW)) * _CW
    npad = rpt * _SUBCORES

    ept = -(-e // (_SUBCORES * _CW)) * _CW
    nb = ept // _CW
    epad = ept * _SUBCORES
    src = jnp.zeros((epad,), jnp.int32).at[:e].set(
        edge_index[0].astype(jnp.int32))
    dst = jnp.zeros((epad,), jnp.int32).at[:e].set(
        edge_index[1].astype(jnp.int32))
    ew = jnp.zeros((epad,), jnp.float32).at[:e].set(
        edge_weight.astype(jnp.float32))
    src2d = src.reshape(_SUBCORES, nb, _CW)
    dst2d = dst.reshape(_SUBCORES, nb, _CW)
    ew2d = ew.reshape(_SUBCORES, nb * _CW)

    b1r = b1.reshape(1, -1)
    b2r = b2.reshape(1, -1)
    b3r = b3.reshape(1, -1)

    ones_chunk = jnp.ones((n, _CW), jnp.float32)
    (deg_out,) = _sc_agg_call(1, npad, nb, (ones_chunk,), src2d, dst2d, ew2d)
    degcol = deg_out[:n, 0:1]

    tc0_out = _tc0(degcol, x, rb)
    dinv, xps = tc0_out[0], tuple(tc0_out[1:])

    agg1 = _sc_agg_call(len(xps), npad, nb, xps, src2d, dst2d, ew2d)
    agg1 = tuple(a[:n] for a in agg1)

    h2ps = tuple(_tc1(dinv, xps, agg1, W1, b1r, W2, rb))

    agg2 = _sc_agg_call(len(h2ps), npad, nb, h2ps, src2d, dst2d, ew2d)
    agg2 = tuple(a[:n] for a in agg2)

    h3ps = tuple(_tc2(dinv, h2ps, agg2, b2r, W3, rb))

    agg3 = _sc_agg_call(len(h3ps), npad, nb, h3ps, src2d, dst2d, ew2d)
    agg3 = tuple(a[:n] for a in agg3)

    out3 = _tc3(dinv, h3ps, agg3, b3r, rb)
    return out3


# speedup vs baseline: 6.4250x; 1.0037x over previous
import functools

import jax
import jax.numpy as jnp
from jax import lax
from jax.experimental import pallas as pl
from jax.experimental.pallas import tpu as pltpu
from jax.experimental.pallas import tpu_sc as plsc

_LANES = 16
_SUBCORES = 16
_CORES = 2
_CW = 128
_BS = 64


@functools.partial(jax.jit, static_argnums=(0, 1, 2))
def _sc_agg_call(n_chunks, npad, nbatch, chunks, src1d, dst2d, ew1d):
    rpt = npad // _SUBCORES

    mesh = plsc.VectorSubcoreMesh(
        core_axis_name="c", subcore_axis_name="s",
        num_cores=_CORES, num_subcores=_SUBCORES)

    out_type = [jax.ShapeDtypeStruct((npad, _CW), jnp.float32)
                for _ in range(n_chunks)]
    nbuf = 2
    scratch_types = (
        [pltpu.VMEM_SHARED((npad, _CW), jnp.float32)]
        + [pltpu.VMEM((nbatch, _BS), jnp.int32)]
        + [pltpu.VMEM((_BS,), jnp.int32)] * nbuf
        + [pltpu.VMEM((_BS,), jnp.float32)] * nbuf
        + [pltpu.VMEM((_BS, _CW), jnp.float32)] * nbuf
        + [pltpu.SemaphoreType.DMA] * (4 * nbuf)
    )

    @functools.partial(pl.kernel, out_type=out_type, mesh=mesh,
                       scratch_types=scratch_types)
    def k(*refs):
        chunk_refs = refs[0:n_chunks]
        src_hbm = refs[n_chunks]
        dst_hbm = refs[n_chunks + 1]
        ew_hbm = refs[n_chunks + 2]
        out_refs = refs[n_chunks + 3: 2 * n_chunks + 3]
        sc = refs[2 * n_chunks + 3:]
        slab = sc[0]
        dst_v = sc[1]
        srcr = sc[2:2 + nbuf]
        ewr = sc[2 + nbuf:2 + 2 * nbuf]
        bufs = sc[2 + 2 * nbuf:2 + 3 * nbuf]
        semg = sc[2 + 3 * nbuf:2 + 4 * nbuf]
        sems = sc[2 + 4 * nbuf:2 + 5 * nbuf]
        semi = sc[2 + 5 * nbuf:2 + 6 * nbuf]
        semw = sc[2 + 6 * nbuf:2 + 7 * nbuf]

        cid = lax.axis_index("c")
        sid = lax.axis_index("s")

        pltpu.sync_copy(dst_hbm.at[sid], dst_v)

        zeros16 = jnp.zeros((_LANES,), jnp.float32)

        def src_load(j, b):
            pltpu.async_copy(src_hbm.at[sid, pl.ds(j * _BS, _BS)],
                             srcr[b], semi[b])

        def ew_load(j, b):
            pltpu.async_copy(ew_hbm.at[sid, pl.ds(j * _BS, _BS)],
                             ewr[b], semw[b])

        def src_wait(b):
            pltpu.make_async_copy(src_hbm.at[sid, pl.ds(0, _BS)],
                                  srcr[b], semi[b]).wait()

        def ew_wait(b):
            pltpu.make_async_copy(ew_hbm.at[sid, pl.ds(0, _BS)],
                                  ewr[b], semw[b]).wait()

        for kk in range(n_chunks):
            @pl.when(cid == (kk % _CORES))
            def _(kk=kk):
                def zrow(i, carry):
                    for c in range(_CW // _LANES):
                        bufs[0][i, pl.ds(c * _LANES, _LANES)] = zeros16
                    return carry
                lax.fori_loop(0, _BS, zrow, 0)
                base = sid * rpt
                for t in range(rpt // _BS):
                    pltpu.sync_copy(
                        bufs[0], slab.at[pl.ds(base + t * _BS, _BS)])
                plsc.subcore_barrier()

                src_load(0, 0)
                src_load(1, 1)
                ew_load(0, 0)
                ew_load(1, 1)
                src_wait(0)
                pltpu.async_copy(chunk_refs[kk].at[srcr[0]], bufs[0],
                                 semg[0])

                def outer(j0, carry):
                    for b in range(nbuf):
                        j = j0 * nbuf + b
                        b2 = (b + 1) % nbuf
                        rows = bufs[b]
                        pltpu.make_async_copy(
                            chunk_refs[kk].at[srcr[b]], rows,
                            semg[b]).wait()

                        @pl.when(j + 2 < nbatch)
                        def _(j=j, b=b):
                            src_load(j + 2, b)

                        ew_wait(b)

                        def group(g, c2, rows=rows, b=b):
                            w16 = ewr[b][pl.ds(g * _LANES, _LANES)]
                            for i2 in range(_LANES):
                                w = w16[i2]
                                r = g * _LANES + i2
                                for c in range(_CW // _LANES):
                                    sl = pl.ds(c * _LANES, _LANES)
                                    rows[r, sl] = rows[r, sl] * w
                            return c2
                        lax.fori_loop(0, _BS // _LANES, group, 0)

                        @pl.when(j + 2 < nbatch)
                        def _(j=j, b=b):
                            ew_load(j + 2, b)

                        pltpu.async_copy(
                            rows, slab.at[dst_v.at[j]], sems[b], add=True)

                        @pl.when(j + 1 < nbatch)
                        def _(j=j, b2=b2):
                            @pl.when(j >= 1)
                            def _():
                                pltpu.make_async_copy(
                                    bufs[b2], slab.at[dst_v.at[j]],
                                    sems[b2]).wait()
                            src_wait(b2)
                            pltpu.async_copy(
                                chunk_refs[kk].at[srcr[b2]], bufs[b2],
                                semg[b2])
                    return carry
                lax.fori_loop(0, nbatch // nbuf, outer, 0)

                for b in range(nbuf):
                    pltpu.make_async_copy(
                        bufs[b], slab.at[dst_v.at[0]], sems[b]).wait()
                plsc.subcore_barrier()

                pltpu.sync_copy(slab.at[pl.ds(base, rpt)],
                                out_refs[kk].at[pl.ds(base, rpt)])

        return None

    return k(*chunks, src1d, dst2d, ew1d)


def _row_spec(rb, w):
    return pl.BlockSpec((rb, w), lambda i: (i, 0))


def _full_spec(shape):
    return pl.BlockSpec(shape, lambda i: tuple(0 for _ in shape))


def _tc0(degcol, x, rb):
    n, din = x.shape
    nck = din // _CW

    def body(deg_ref, x_ref, dinv_ref, *xp_refs):
        deg = deg_ref[...] + 1.0
        dinv = jnp.where(deg > 0, lax.rsqrt(jnp.maximum(deg, 1e-12)), 0.0)
        dinv_ref[...] = dinv
        xp = x_ref[...] * dinv
        for c in range(nck):
            xp_refs[c][...] = xp[:, c * _CW:(c + 1) * _CW]

    return pl.pallas_call(
        body,
        grid=(n // rb,),
        in_specs=[_row_spec(rb, 1), _row_spec(rb, din)],
        out_specs=[_row_spec(rb, 1)] + [_row_spec(rb, _CW)] * nck,
        out_shape=[jax.ShapeDtypeStruct((n, 1), jnp.float32)]
        + [jax.ShapeDtypeStruct((n, _CW), jnp.float32)] * nck,
    )(degcol, x)


def _tc1(dinv, xps, aggs, W1, b1, W2, rb):
    n = dinv.shape[0]
    nin = len(xps)
    dh = W2.shape[1]
    nout = dh // _CW

    def body(*refs):
        dinv_ref = refs[0]
        xp_refs = refs[1:1 + nin]
        ag_refs = refs[1 + nin:1 + 2 * nin]
        W1_ref, b1_ref, W2_ref = refs[1 + 2 * nin:4 + 2 * nin]
        out_refs = refs[4 + 2 * nin:]
        dinv = dinv_ref[...]
        m1 = jnp.concatenate(
            [ag_refs[c][...] + xp_refs[c][...] for c in range(nin)],
            axis=1) * dinv
        out1 = jnp.maximum(
            jnp.dot(m1, W1_ref[...], preferred_element_type=jnp.float32)
            + b1_ref[...], 0.0)
        h2p = jnp.dot(out1, W2_ref[...],
                      preferred_element_type=jnp.float32) * dinv
        for c in range(nout):
            out_refs[c][...] = h2p[:, c * _CW:(c + 1) * _CW]

    return pl.pallas_call(
        body,
        grid=(n // rb,),
        in_specs=[_row_spec(rb, 1)] + [_row_spec(rb, _CW)] * (2 * nin)
        + [_full_spec(W1.shape), _full_spec(b1.shape), _full_spec(W2.shape)],
        out_specs=[_row_spec(rb, _CW)] * nout,
        out_shape=[jax.ShapeDtypeStruct((n, _CW), jnp.float32)] * nout,
    )(dinv, *xps, *aggs, W1, b1, W2)


def _tc2(dinv, h2ps, aggs, b2, W3, rb):
    n = dinv.shape[0]
    nin = len(h2ps)
    dout = W3.shape[1]
    nout = dout // _CW

    def body(*refs):
        dinv_ref = refs[0]
        hp_refs = refs[1:1 + nin]
        ag_refs = refs[1 + nin:1 + 2 * nin]
        b2_ref, W3_ref = refs[1 + 2 * nin:3 + 2 * nin]
        out_refs = refs[3 + 2 * nin:]
        dinv = dinv_ref[...]
        m2 = jnp.concatenate(
            [ag_refs[c][...] + hp_refs[c][...] for c in range(nin)],
            axis=1) * dinv
        out2 = jnp.maximum(m2 + b2_ref[...], 0.0)
        h3p = jnp.dot(out2, W3_ref[...],
                      preferred_element_type=jnp.float32) * dinv
        for c in range(nout):
            out_refs[c][...] = h3p[:, c * _CW:(c + 1) * _CW]

    return pl.pallas_call(
        body,
        grid=(n // rb,),
        in_specs=[_row_spec(rb, 1)] + [_row_spec(rb, _CW)] * (2 * nin)
        + [_full_spec(b2.shape), _full_spec(W3.shape)],
        out_specs=[_row_spec(rb, _CW)] * nout,
        out_shape=[jax.ShapeDtypeStruct((n, _CW), jnp.float32)] * nout,
    )(dinv, *h2ps, *aggs, b2, W3)


def _tc3(dinv, h3ps, aggs, b3, rb):
    n = dinv.shape[0]
    nin = len(h3ps)
    dout = nin * _CW

    def body(*refs):
        dinv_ref = refs[0]
        hp_refs = refs[1:1 + nin]
        ag_refs = refs[1 + nin:1 + 2 * nin]
        b3_ref = refs[1 + 2 * nin]
        out_ref = refs[2 + 2 * nin]
        dinv = dinv_ref[...]
        m3 = jnp.concatenate(
            [ag_refs[c][...] + hp_refs[c][...] for c in range(nin)],
            axis=1) * dinv
        out_ref[...] = m3 + b3_ref[...]

    return pl.pallas_call(
        body,
        grid=(n // rb,),
        in_specs=[_row_spec(rb, 1)] + [_row_spec(rb, _CW)] * (2 * nin)
        + [_full_spec(b3.shape)],
        out_specs=_row_spec(rb, dout),
        out_shape=jax.ShapeDtypeStruct((n, dout), jnp.float32),
    )(dinv, *h3ps, *aggs, b3)


def kernel(x, edge_index, edge_weight, W1, b1, W2, b2, W3, b3):
    n, din = x.shape
    e = edge_index.shape[1]
    dh = W1.shape[1]
    dout = W3.shape[1]

    rb = 1000 if n % 1000 == 0 else 8 * (n // 8)
    rpt = -(-n // (_SUBCORES * _CW)) * _CW
    npad = rpt * _SUBCORES

    ept = -(-e // (_SUBCORES * 2 * _BS)) * 2 * _BS
    nbatch = ept // _BS
    epad = ept * _SUBCORES
    src = jnp.zeros((epad,), jnp.int32).at[:e].set(
        edge_index[0].astype(jnp.int32))
    dst = jnp.zeros((epad,), jnp.int32).at[:e].set(
        edge_index[1].astype(jnp.int32))
    ew = jnp.zeros((epad,), jnp.float32).at[:e].set(
        edge_weight.astype(jnp.float32))
    src1d = src.reshape(_SUBCORES, nbatch * _BS)
    dst2d = dst.reshape(_SUBCORES, nbatch, _BS)
    ew1d = ew.reshape(_SUBCORES, nbatch * _BS)

    b1r = b1.reshape(1, -1)
    b2r = b2.reshape(1, -1)
    b3r = b3.reshape(1, -1)

    ones_chunk = jnp.ones((n, _CW), jnp.float32)
    (deg_out,) = _sc_agg_call(1, npad, nbatch, (ones_chunk,), src1d, dst2d, ew1d)
    degcol = deg_out[:n, 0:1]

    tc0_out = _tc0(degcol, x, rb)
    dinv, xps = tc0_out[0], tuple(tc0_out[1:])

    agg1 = _sc_agg_call(len(xps), npad, nbatch, xps, src1d, dst2d, ew1d)
    agg1 = tuple(a[:n] for a in agg1)

    h2ps = tuple(_tc1(dinv, xps, agg1, W1, b1r, W2, rb))

    agg2 = _sc_agg_call(len(h2ps), npad, nbatch, h2ps, src1d, dst2d, ew1d)
    agg2 = tuple(a[:n] for a in agg2)

    h3ps = tuple(_tc2(dinv, h2ps, agg2, b2r, W3, rb))

    agg3 = _sc_agg_call(len(h3ps), npad, nbatch, h3ps, src1d, dst2d, ew1d)
    agg3 = tuple(a[:n] for a in agg3)

    out3 = _tc3(dinv, h3ps, agg3, b3r, rb)
    return out3


# speedup vs baseline: 6.5521x; 1.0198x over previous
import functools

import jax
import jax.numpy as jnp
from jax import lax
from jax.experimental import pallas as pl
from jax.experimental.pallas import tpu as pltpu
from jax.experimental.pallas import tpu_sc as plsc

_LANES = 16
_SUBCORES = 16
_CORES = 2
_CW = 128
_BS = 64


@functools.partial(jax.jit, static_argnums=(0, 1, 2, 3))
def _sc_agg_call(n_chunks, npad, nbatch, ranges, chunks, src1d, dst1d, ew1d):
    rpt = npad // _SUBCORES

    mesh = plsc.VectorSubcoreMesh(
        core_axis_name="c", subcore_axis_name="s",
        num_cores=_CORES, num_subcores=_SUBCORES)

    out_type = [jax.ShapeDtypeStruct((npad, _CW), jnp.float32)
                for _ in range(n_chunks)]
    nbuf = 4
    scratch_types = (
        [pltpu.VMEM_SHARED((npad, _CW), jnp.float32)]
        + [pltpu.VMEM((_BS,), jnp.int32)] * nbuf
        + [pltpu.VMEM((_BS,), jnp.int32)] * nbuf
        + [pltpu.VMEM((_BS,), jnp.float32)] * nbuf
        + [pltpu.VMEM((_BS, _CW), jnp.float32)] * nbuf
        + [pltpu.SemaphoreType.DMA] * (5 * nbuf)
    )

    @functools.partial(pl.kernel, out_type=out_type, mesh=mesh,
                       scratch_types=scratch_types)
    def k(*refs):
        chunk_refs = refs[0:n_chunks]
        src_hbm = refs[n_chunks]
        dst_hbm = refs[n_chunks + 1]
        ew_hbm = refs[n_chunks + 2]
        out_refs = refs[n_chunks + 3: 2 * n_chunks + 3]
        sc = refs[2 * n_chunks + 3:]
        slab = sc[0]
        srcr = sc[1:1 + nbuf]
        dstr = sc[1 + nbuf:1 + 2 * nbuf]
        ewr = sc[1 + 2 * nbuf:1 + 3 * nbuf]
        bufs = sc[1 + 3 * nbuf:1 + 4 * nbuf]
        semg = sc[1 + 4 * nbuf:1 + 5 * nbuf]
        sems = sc[1 + 5 * nbuf:1 + 6 * nbuf]
        semi = sc[1 + 6 * nbuf:1 + 7 * nbuf]
        semw = sc[1 + 7 * nbuf:1 + 8 * nbuf]
        semd = sc[1 + 8 * nbuf:1 + 9 * nbuf]

        cid = lax.axis_index("c")
        sid = lax.axis_index("s")

        zeros16 = jnp.zeros((_LANES,), jnp.float32)

        def load(hbm, j, ring, sem, b):
            pltpu.async_copy(hbm.at[sid, pl.ds(j * _BS, _BS)],
                             ring[b], sem[b])

        def load_wait(hbm, ring, sem, b):
            pltpu.make_async_copy(hbm.at[sid, pl.ds(0, _BS)],
                                  ring[b], sem[b]).wait()

        for kk in range(n_chunks):
            lo, cnt = ranges[kk]

            @pl.when(cid == (kk % _CORES))
            def _(kk=kk, lo=lo, cnt=cnt):
                def zrow(i, carry):
                    for c in range(_CW // _LANES):
                        bufs[0][i, pl.ds(c * _LANES, _LANES)] = zeros16
                    return carry
                lax.fori_loop(0, _BS, zrow, 0)
                base = sid * rpt
                for t in range(rpt // _BS):
                    pltpu.sync_copy(
                        bufs[0], slab.at[pl.ds(base + t * _BS, _BS)])
                plsc.subcore_barrier()

                for t0 in range(nbuf):
                    load(src_hbm, lo + t0, srcr, semi, t0)
                    load(ew_hbm, lo + t0, ewr, semw, t0)
                    load(dst_hbm, lo + t0, dstr, semd, t0)
                load_wait(src_hbm, srcr, semi, 0)
                pltpu.async_copy(chunk_refs[kk].at[srcr[0]], bufs[0],
                                 semg[0])
                load_wait(src_hbm, srcr, semi, 1)
                pltpu.async_copy(chunk_refs[kk].at[srcr[1]], bufs[1],
                                 semg[1])

                def outer(tt, carry):
                    for b in range(nbuf):
                        t = tt * nbuf + b
                        j = lo + t
                        b2 = (b + 2) % nbuf
                        rows = bufs[b]
                        pltpu.make_async_copy(
                            chunk_refs[kk].at[srcr[b]], rows,
                            semg[b]).wait()

                        @pl.when(t + nbuf < cnt)
                        def _(j=j, b=b):
                            load(src_hbm, j + nbuf, srcr, semi, b)

                        load_wait(ew_hbm, ewr, semw, b)

                        def group(g, c2, rows=rows, b=b):
                            w16 = ewr[b][pl.ds(g * _LANES, _LANES)]
                            for i2 in range(_LANES):
                                w = w16[i2]
                                r = g * _LANES + i2
                                for c in range(_CW // _LANES):
                                    sl = pl.ds(c * _LANES, _LANES)
                                    rows[r, sl] = rows[r, sl] * w
                            return c2
                        lax.fori_loop(0, _BS // _LANES, group, 0)

                        @pl.when(t + nbuf < cnt)
                        def _(j=j, b=b):
                            load(ew_hbm, j + nbuf, ewr, semw, b)

                        load_wait(dst_hbm, dstr, semd, b)
                        pltpu.async_copy(
                            rows, slab.at[dstr[b]], sems[b], add=True)

                        @pl.when(t + 2 < cnt)
                        def _(t=t, j=j, b2=b2):
                            @pl.when(t >= 2)
                            def _():
                                pltpu.make_async_copy(
                                    bufs[b2], slab.at[dstr[b2]],
                                    sems[b2]).wait()
                                load(dst_hbm, j + 2, dstr, semd, b2)
                            load_wait(src_hbm, srcr, semi, b2)
                            pltpu.async_copy(
                                chunk_refs[kk].at[srcr[b2]], bufs[b2],
                                semg[b2])
                    return carry
                lax.fori_loop(0, cnt // nbuf, outer, 0)

                for b in range(nbuf):
                    pltpu.make_async_copy(
                        bufs[b], slab.at[dstr[b]], sems[b]).wait()
                plsc.subcore_barrier()

                pltpu.sync_copy(slab.at[pl.ds(base, rpt)],
                                out_refs[kk].at[pl.ds(base, rpt)])

        return None

    return k(*chunks, src1d, dst1d, ew1d)


def _row_spec(rb, w):
    return pl.BlockSpec((rb, w), lambda i: (i, 0))


def _full_spec(shape):
    return pl.BlockSpec(shape, lambda i: tuple(0 for _ in shape))


def _tc0(degcols, x, rb):
    n, din = x.shape
    nck = din // _CW

    def body(dega_ref, degb_ref, x_ref, dinv_ref, *xp_refs):
        deg = dega_ref[...] + degb_ref[...] + 1.0
        dinv = jnp.where(deg > 0, lax.rsqrt(jnp.maximum(deg, 1e-12)), 0.0)
        dinv_ref[...] = dinv
        xp = x_ref[...] * dinv
        for c in range(nck):
            xp_refs[c][...] = xp[:, c * _CW:(c + 1) * _CW]

    return pl.pallas_call(
        body,
        grid=(n // rb,),
        in_specs=[_row_spec(rb, 1), _row_spec(rb, 1), _row_spec(rb, din)],
        out_specs=[_row_spec(rb, 1)] + [_row_spec(rb, _CW)] * nck,
        out_shape=[jax.ShapeDtypeStruct((n, 1), jnp.float32)]
        + [jax.ShapeDtypeStruct((n, _CW), jnp.float32)] * nck,
    )(*degcols, x)


def _tc1(dinv, xps, aggs, W1, b1, W2, rb):
    n = dinv.shape[0]
    nin = len(xps)
    dh = W2.shape[1]
    nout = dh // _CW

    def body(*refs):
        dinv_ref = refs[0]
        xp_refs = refs[1:1 + nin]
        ag_refs = refs[1 + nin:1 + 2 * nin]
        W1_ref, b1_ref, W2_ref = refs[1 + 2 * nin:4 + 2 * nin]
        out_refs = refs[4 + 2 * nin:]
        dinv = dinv_ref[...]
        m1 = jnp.concatenate(
            [ag_refs[c][...] + xp_refs[c][...] for c in range(nin)],
            axis=1) * dinv
        out1 = jnp.maximum(
            jnp.dot(m1, W1_ref[...], preferred_element_type=jnp.float32)
            + b1_ref[...], 0.0)
        h2p = jnp.dot(out1, W2_ref[...],
                      preferred_element_type=jnp.float32) * dinv
        for c in range(nout):
            out_refs[c][...] = h2p[:, c * _CW:(c + 1) * _CW]

    return pl.pallas_call(
        body,
        grid=(n // rb,),
        in_specs=[_row_spec(rb, 1)] + [_row_spec(rb, _CW)] * (2 * nin)
        + [_full_spec(W1.shape), _full_spec(b1.shape), _full_spec(W2.shape)],
        out_specs=[_row_spec(rb, _CW)] * nout,
        out_shape=[jax.ShapeDtypeStruct((n, _CW), jnp.float32)] * nout,
    )(dinv, *xps, *aggs, W1, b1, W2)


def _tc2(dinv, h2ps, aggs, b2, W3, rb):
    n = dinv.shape[0]
    nin = len(h2ps)
    dout = W3.shape[1]
    nout = dout // _CW

    def body(*refs):
        dinv_ref = refs[0]
        hp_refs = refs[1:1 + nin]
        ag_refs = refs[1 + nin:1 + 2 * nin]
        b2_ref, W3_ref = refs[1 + 2 * nin:3 + 2 * nin]
        out_refs = refs[3 + 2 * nin:]
        dinv = dinv_ref[...]
        m2 = jnp.concatenate(
            [ag_refs[c][...] + hp_refs[c][...] for c in range(nin)],
            axis=1) * dinv
        out2 = jnp.maximum(m2 + b2_ref[...], 0.0)
        h3p = jnp.dot(out2, W3_ref[...],
                      preferred_element_type=jnp.float32) * dinv
        for c in range(nout):
            out_refs[c][...] = h3p[:, c * _CW:(c + 1) * _CW]

    return pl.pallas_call(
        body,
        grid=(n // rb,),
        in_specs=[_row_spec(rb, 1)] + [_row_spec(rb, _CW)] * (2 * nin)
        + [_full_spec(b2.shape), _full_spec(W3.shape)],
        out_specs=[_row_spec(rb, _CW)] * nout,
        out_shape=[jax.ShapeDtypeStruct((n, _CW), jnp.float32)] * nout,
    )(dinv, *h2ps, *aggs, b2, W3)


def _tc3(dinv, h3ps, aggs, b3, rb):
    n = dinv.shape[0]
    nin = len(h3ps)
    dout = nin * _CW

    def body(*refs):
        dinv_ref = refs[0]
        hp_refs = refs[1:1 + nin]
        ag_refs = refs[1 + nin:1 + 2 * nin]
        b3_ref = refs[1 + 2 * nin]
        out_ref = refs[2 + 2 * nin]
        dinv = dinv_ref[...]
        m3 = jnp.concatenate(
            [ag_refs[c][...] + hp_refs[c][...] for c in range(nin)],
            axis=1) * dinv
        out_ref[...] = m3 + b3_ref[...]

    return pl.pallas_call(
        body,
        grid=(n // rb,),
        in_specs=[_row_spec(rb, 1)] + [_row_spec(rb, _CW)] * (2 * nin)
        + [_full_spec(b3.shape)],
        out_specs=_row_spec(rb, dout),
        out_shape=jax.ShapeDtypeStruct((n, dout), jnp.float32),
    )(dinv, *h3ps, *aggs, b3)


def kernel(x, edge_index, edge_weight, W1, b1, W2, b2, W3, b3):
    n, din = x.shape
    e = edge_index.shape[1]
    dh = W1.shape[1]
    dout = W3.shape[1]

    rb = 1000 if n % 1000 == 0 else 8 * (n // 8)
    rpt = -(-n // (_SUBCORES * _CW)) * _CW
    npad = rpt * _SUBCORES

    ept = -(-e // (_SUBCORES * 8 * _BS)) * 8 * _BS
    nbatch = ept // _BS
    epad = ept * _SUBCORES
    src = jnp.zeros((epad,), jnp.int32).at[:e].set(
        edge_index[0].astype(jnp.int32))
    dst = jnp.zeros((epad,), jnp.int32).at[:e].set(
        edge_index[1].astype(jnp.int32))
    ew = jnp.zeros((epad,), jnp.float32).at[:e].set(
        edge_weight.astype(jnp.float32))
    src1d = src.reshape(_SUBCORES, nbatch * _BS)
    dst1d = dst.reshape(_SUBCORES, nbatch * _BS)
    ew1d = ew.reshape(_SUBCORES, nbatch * _BS)

    b1r = b1.reshape(1, -1)
    b2r = b2.reshape(1, -1)
    b3r = b3.reshape(1, -1)

    full = ((0, nbatch),)

    ones_chunk = jnp.ones((n, _CW), jnp.float32)
    half = nbatch // 2
    dega, degb = _sc_agg_call(2, npad, nbatch, ((0, half), (half, half)),
                              (ones_chunk, ones_chunk), src1d, dst1d, ew1d)
    degcols = (dega[:n, 0:1], degb[:n, 0:1])

    tc0_out = _tc0(degcols, x, rb)
    dinv, xps = tc0_out[0], tuple(tc0_out[1:])

    agg1 = _sc_agg_call(len(xps), npad, nbatch, full * len(xps),
                        xps, src1d, dst1d, ew1d)
    agg1 = tuple(a[:n] for a in agg1)

    h2ps = tuple(_tc1(dinv, xps, agg1, W1, b1r, W2, rb))

    agg2 = _sc_agg_call(len(h2ps), npad, nbatch, full * len(h2ps),
                        h2ps, src1d, dst1d, ew1d)
    agg2 = tuple(a[:n] for a in agg2)

    h3ps = tuple(_tc2(dinv, h2ps, agg2, b2r, W3, rb))

    agg3 = _sc_agg_call(len(h3ps), npad, nbatch, full * len(h3ps),
                        h3ps, src1d, dst1d, ew1d)
    agg3 = tuple(a[:n] for a in agg3)

    out3 = _tc3(dinv, h3ps, agg3, b3r, rb)
    return out3
